# baseline (speedup 1.0000x reference)
import functools

import jax
import jax.numpy as jnp
from jax import lax
from jax.experimental import pallas as pl
from jax.experimental.pallas import tpu as pltpu
from jax.experimental.pallas import tpu_sc as plsc

_N_HEADS = 16
_MAX_DIST = 512
_Q = 2048
_K = 2048
_P_LEFT = _K - _MAX_DIST - 1
_E_LEN = 4096
_ROWS_PER_W = 1024
_GROUPS = _ROWS_PER_W // 8
_FIRE = 8


def _sc_expand_body(e8_hbm, out_hbm, e8_v, sem):
    wid = lax.axis_index("s") * 2 + lax.axis_index("c")
    h = wid // 2
    row_base = (wid % 2) * _ROWS_PER_W
    pltpu.sync_copy(e8_hbm.at[h], e8_v)

    def body(g, carry):
        copies = []
        for b in range(_FIRE):
            i = row_base + _FIRE * g + b
            r = i % 8
            off = pl.multiple_of(r * _E_LEN + _K - (i - r), 8)
            dst = pl.multiple_of((h * _Q + i) * _K, 128)
            copies.append(pltpu.make_async_copy(
                e8_v.at[pl.ds(off, _K)],
                out_hbm.at[pl.ds(dst, _K)],
                sem))
        for cp in copies:
            cp.start()
        for cp in copies:
            cp.wait()
        return carry

    lax.fori_loop(0, _ROWS_PER_W // _FIRE, body, 0)


def kernel(q_len, k_len, rel_bias):
    d = (k_len - _K) - (q_len - _Q)
    vlen = _E_LEN + 7
    w = jnp.arange(vlen)[:, None]
    mid_start = 8 + _P_LEFT - d
    base = jnp.where(w < mid_start, rel_bias[0][None, :], rel_bias[-1][None, :])
    v8 = lax.dynamic_update_slice(base, rel_bias, (mid_start, 0))
    e8 = jnp.stack([lax.slice(v8, (7 - r, 0), (7 - r + _E_LEN, _N_HEADS))
                    for r in range(8)])
    e8 = jnp.transpose(e8, (2, 0, 1)).reshape(_N_HEADS, 8 * _E_LEN)

    sc_expand = functools.partial(
        pl.kernel,
        mesh=plsc.VectorSubcoreMesh(core_axis_name="c", subcore_axis_name="s"),
        out_type=jax.ShapeDtypeStruct((_N_HEADS * _Q * _K,), rel_bias.dtype),
        scratch_types=[
            pltpu.VMEM((8 * _E_LEN,), rel_bias.dtype),
            pltpu.SemaphoreType.DMA,
        ],
    )(_sc_expand_body)
    return sc_expand(e8).reshape(_N_HEADS, _Q, _K)


# speedup vs baseline: 3.2677x; 3.2677x over previous
import functools

import jax
import jax.numpy as jnp
from jax import lax
from jax.experimental import pallas as pl
from jax.experimental.pallas import tpu as pltpu
from jax.experimental.pallas import tpu_sc as plsc

_N_HEADS = 16
_MAX_DIST = 512
_TBL = 2 * _MAX_DIST + 1
_Q = 2048
_K = 2048
_P_LEFT = _K - _MAX_DIST - 1
_E8_LEN = 4224
_E128_LEN = 4096
_U_VECS = _E8_LEN // 16


def _sc_prep_body(tbl_hbm, dvec_hbm, e8_hbm, tbl_v, dvec_v, e8s_v, sem):
    wid = lax.axis_index("s") * 2 + lax.axis_index("c")
    h = wid // 2
    half = wid % 2
    pltpu.sync_copy(tbl_hbm, tbl_v)
    pltpu.sync_copy(dvec_hbm, dvec_v)
    dvec = dvec_v[...]
    iota = lax.iota(jnp.int32, 16)

    def body(t, carry):
        r_loc = t // _U_VECS
        u_base = (t % _U_VECS) * 16
        r = 4 * half + r_loc
        idx = jnp.clip(u_base + iota - r - _P_LEFT + dvec, 0, _TBL - 1) * _N_HEADS + h
        vals = plsc.load_gather(tbl_v, [idx])
        e8s_v[pl.ds(pl.multiple_of(r_loc * _E8_LEN + u_base, 8), 16)] = vals
        return carry

    lax.fori_loop(0, 4 * _U_VECS, body, 0)
    dst = pl.multiple_of((h * 8 + 4 * half) * _E8_LEN, 128)
    pltpu.sync_copy(e8s_v, e8_hbm.at[pl.ds(dst, 4 * _E8_LEN)])


def _expand_body(e8_ref, out_ref, e128_ref):
    for a in range(16):
        off = 127 - 8 * a
        e128_ref[8 * a:8 * a + 8, :] = e8_ref[0, :, off:off + _E128_LEN]
    for b in range(16):
        s = 1920 - 128 * b
        out_ref[0, 128 * b:128 * (b + 1), :] = e128_ref[:, s:s + _K]


def kernel(q_len, k_len, rel_bias):
    d = (k_len - _K) - (q_len - _Q)

    sc_prep = functools.partial(
        pl.kernel,
        mesh=plsc.VectorSubcoreMesh(core_axis_name="c", subcore_axis_name="s"),
        out_type=jax.ShapeDtypeStruct((_N_HEADS * 8 * _E8_LEN,), rel_bias.dtype),
        scratch_types=[
            pltpu.VMEM((_TBL * _N_HEADS,), rel_bias.dtype),
            pltpu.VMEM((16,), jnp.int32),
            pltpu.VMEM((4 * _E8_LEN,), rel_bias.dtype),
            pltpu.SemaphoreType.DMA,
        ],
        compiler_params=pltpu.CompilerParams(needs_layout_passes=False),
    )(_sc_prep_body)
    e8 = sc_prep(rel_bias.reshape(-1),
                 jnp.full((16,), d, jnp.int32)).reshape(_N_HEADS, 8, _E8_LEN)

    out = pl.pallas_call(
        _expand_body,
        grid=(_N_HEADS,),
        in_specs=[pl.BlockSpec((1, 8, _E8_LEN), lambda h: (h, 0, 0))],
        out_specs=pl.BlockSpec((1, _Q, _K), lambda h: (h, 0, 0)),
        out_shape=jax.ShapeDtypeStruct((_N_HEADS, _Q, _K), rel_bias.dtype),
        scratch_shapes=[pltpu.VMEM((128, _E128_LEN), rel_bias.dtype)],
        compiler_params=pltpu.CompilerParams(
            dimension_semantics=("parallel",),
        ),
    )(e8)
    return out


# speedup vs baseline: 3.2735x; 1.0018x over previous
import functools

import jax
import jax.numpy as jnp
from jax import lax
from jax.experimental import pallas as pl
from jax.experimental.pallas import tpu as pltpu
from jax.experimental.pallas import tpu_sc as plsc

_N_HEADS = 16
_MAX_DIST = 512
_TBL = 2 * _MAX_DIST + 1
_Q = 2048
_K = 2048
_P_LEFT = _K - _MAX_DIST - 1
_E8_LEN = 4224
_E128_LEN = 4096
_U_VECS = _E8_LEN // 16


def _sc_prep_body(tbl_hbm, dvec_hbm, e8_hbm, tbl_v, dvec_v, e8s_v, sem):
    wid = lax.axis_index("s") * 2 + lax.axis_index("c")
    h = wid // 2
    half = wid % 2
    pltpu.sync_copy(tbl_hbm, tbl_v)
    pltpu.sync_copy(dvec_hbm, dvec_v)
    dvec = dvec_v[...]
    iota = lax.iota(jnp.int32, 16)
    unroll = 8
    for r_loc in range(4):
        r = 4 * half + r_loc
        base_vec = iota - r - _P_LEFT + dvec

        def body(t, carry, r_loc=r_loc, base_vec=base_vec):
            u0 = t * (16 * unroll)
            for k in range(unroll):
                u_base = u0 + 16 * k
                idx = jnp.clip(u_base + base_vec, 0, _TBL - 1) * _N_HEADS + h
                vals = plsc.load_gather(tbl_v, [idx])
                e8s_v[pl.ds(pl.multiple_of(r_loc * _E8_LEN + u_base, 8), 16)] = vals
            return carry

        lax.fori_loop(0, _U_VECS // unroll, body, 0)
    dst = pl.multiple_of((h * 8 + 4 * half) * _E8_LEN, 128)
    pltpu.sync_copy(e8s_v, e8_hbm.at[pl.ds(dst, 4 * _E8_LEN)])


def _expand_body(e8_ref, out_ref, e128_ref):
    for a in range(16):
        off = 127 - 8 * a
        e128_ref[8 * a:8 * a + 8, :] = e8_ref[0, :, off:off + _E128_LEN]
    for b in range(16):
        s = 1920 - 128 * b
        out_ref[0, 128 * b:128 * (b + 1), :] = e128_ref[:, s:s + _K]


def kernel(q_len, k_len, rel_bias):
    d = (k_len - _K) - (q_len - _Q)

    sc_prep = functools.partial(
        pl.kernel,
        mesh=plsc.VectorSubcoreMesh(core_axis_name="c", subcore_axis_name="s"),
        out_type=jax.ShapeDtypeStruct((_N_HEADS * 8 * _E8_LEN,), rel_bias.dtype),
        scratch_types=[
            pltpu.VMEM((_TBL * _N_HEADS,), rel_bias.dtype),
            pltpu.VMEM((16,), jnp.int32),
            pltpu.VMEM((4 * _E8_LEN,), rel_bias.dtype),
            pltpu.SemaphoreType.DMA,
        ],
        compiler_params=pltpu.CompilerParams(needs_layout_passes=False),
    )(_sc_prep_body)
    e8 = sc_prep(rel_bias.reshape(-1),
                 jnp.full((16,), d, jnp.int32)).reshape(_N_HEADS, 8, _E8_LEN)

    out = pl.pallas_call(
        _expand_body,
        grid=(_N_HEADS,),
        in_specs=[pl.BlockSpec((1, 8, _E8_LEN), lambda h: (h, 0, 0))],
        out_specs=pl.BlockSpec((1, _Q, _K), lambda h: (h, 0, 0)),
        out_shape=jax.ShapeDtypeStruct((_N_HEADS, _Q, _K), rel_bias.dtype),
        scratch_shapes=[pltpu.VMEM((128, _E128_LEN), rel_bias.dtype)],
        compiler_params=pltpu.CompilerParams(
            dimension_semantics=("parallel",),
        ),
    )(e8)
    return out


# speedup vs baseline: 3.3121x; 1.0118x over previous
import functools

import jax
import jax.numpy as jnp
from jax import lax
from jax.experimental import pallas as pl
from jax.experimental.pallas import tpu as pltpu
from jax.experimental.pallas import tpu_sc as plsc

_N_HEADS = 16
_MAX_DIST = 512
_TBL = 2 * _MAX_DIST + 1
_Q = 2048
_K = 2048
_P_LEFT = _K - _MAX_DIST - 1
_E8_LEN = 4224
_E128_LEN = 4096
_U_VECS = _E8_LEN // 16


def _sc_prep_body(tbl_hbm, e8_hbm, tbl_v, e8s_v, sem):
    wid = lax.axis_index("s") * 2 + lax.axis_index("c")
    h = wid // 2
    half = wid % 2
    pltpu.sync_copy(tbl_hbm, tbl_v)
    dvec = tbl_v[pl.ds(_TBL * _N_HEADS, 16)].astype(jnp.int32)
    iota = lax.iota(jnp.int32, 16)
    unroll = 8
    for r_loc in range(4):
        r = 4 * half + r_loc
        base_vec = iota - r - _P_LEFT + dvec

        def body(t, carry, r_loc=r_loc, base_vec=base_vec):
            u0 = t * (16 * unroll)
            for k in range(unroll):
                u_base = u0 + 16 * k
                idx = jnp.clip(u_base + base_vec, 0, _TBL - 1) * _N_HEADS + h
                vals = plsc.load_gather(tbl_v, [idx])
                e8s_v[pl.ds(pl.multiple_of(r_loc * _E8_LEN + u_base, 8), 16)] = vals
            return carry

        lax.fori_loop(0, _U_VECS // unroll, body, 0)
    dst = pl.multiple_of((h * 8 + 4 * half) * _E8_LEN, 128)
    pltpu.sync_copy(e8s_v, e8_hbm.at[pl.ds(dst, 4 * _E8_LEN)])


def _expand_body(e8_ref, out_ref, e128_ref):
    for a in range(16):
        off = 127 - 8 * a
        e128_ref[8 * a:8 * a + 8, :] = e8_ref[0, :, off:off + _E128_LEN]
    for b in range(16):
        s = 1920 - 128 * b
        out_ref[0, 128 * b:128 * (b + 1), :] = e128_ref[:, s:s + _K]


def kernel(q_len, k_len, rel_bias):
    d = (k_len - _K) - (q_len - _Q)

    sc_prep = functools.partial(
        pl.kernel,
        mesh=plsc.VectorSubcoreMesh(core_axis_name="c", subcore_axis_name="s"),
        out_type=jax.ShapeDtypeStruct((_N_HEADS * 8 * _E8_LEN,), rel_bias.dtype),
        scratch_types=[
            pltpu.VMEM((_TBL * _N_HEADS + 16,), rel_bias.dtype),
            pltpu.VMEM((4 * _E8_LEN,), rel_bias.dtype),
            pltpu.SemaphoreType.DMA,
        ],
        compiler_params=pltpu.CompilerParams(needs_layout_passes=False),
    )(_sc_prep_body)
    tbl_plus = jnp.concatenate(
        [rel_bias.reshape(-1), jnp.full((16,), d, rel_bias.dtype)])
    e8 = sc_prep(tbl_plus).reshape(_N_HEADS, 8, _E8_LEN)

    out = pl.pallas_call(
        _expand_body,
        grid=(_N_HEADS,),
        in_specs=[pl.BlockSpec((1, 8, _E8_LEN), lambda h: (h, 0, 0))],
        out_specs=pl.BlockSpec((1, _Q, _K), lambda h: (h, 0, 0)),
        out_shape=jax.ShapeDtypeStruct((_N_HEADS, _Q, _K), rel_bias.dtype),
        scratch_shapes=[pltpu.VMEM((128, _E128_LEN), rel_bias.dtype)],
        compiler_params=pltpu.CompilerParams(
            dimension_semantics=("parallel",),
        ),
    )(e8)
    return out
